# baseline (speedup 1.0000x reference)
import functools
import jax
import jax.numpy as jnp
from jax import lax
from jax.experimental import pallas as pl
from jax.experimental.pallas import tpu as pltpu
from jax.experimental.pallas import tpu_sc as plsc

_E = 64
_ROWS = 1024
_H = 2048

_TSH = 8
_HSH = 4
_HL = _H // _HSH
_TCHUNK = 64


def _router_body(x_ref, wt_ref, rb_ref, meta_ref):
    x = x_ref[...]
    logits = jnp.dot(x, wt_ref[...], preferred_element_type=jnp.float32)
    logits = logits + rb_ref[...]
    idx = lax.broadcasted_iota(jnp.int32, logits.shape, 1)
    ml1 = jnp.max(logits, axis=1, keepdims=True)
    i1 = jnp.min(jnp.where(logits == ml1, idx, _E), axis=1, keepdims=True)
    lm = jnp.where(idx == i1, -jnp.inf, logits)
    ml2 = jnp.max(lm, axis=1, keepdims=True)
    i2 = jnp.min(jnp.where(lm == ml2, idx, _E), axis=1, keepdims=True)
    ex = jnp.exp(logits - ml1)
    r = 1.0 / jnp.sum(ex, axis=1, keepdims=True)
    w1 = jnp.exp(ml2 - ml1) * r
    w0 = r
    ones = jnp.ones((1, 16), jnp.float32)
    meta_ref[...] = jnp.concatenate(
        [(w0 + w1) * ones, w0 * ones, w1 * ones,
         (i1 * _HL).astype(jnp.float32) * ones,
         (i2 * _HL).astype(jnp.float32) * ones], axis=1)


def _router(flat, router_weight, router_bias):
    BS = flat.shape[0]
    wt = router_weight.T
    rb = router_bias.reshape(1, _E)
    return pl.pallas_call(
        _router_body,
        grid=(BS // _ROWS,),
        in_specs=[
            pl.BlockSpec((_ROWS, _H), lambda i: (i, 0)),
            pl.BlockSpec((_H, _E), lambda i: (0, 0)),
            pl.BlockSpec((1, _E), lambda i: (0, 0)),
        ],
        out_specs=pl.BlockSpec((_ROWS, 80), lambda i: (i, 0)),
        out_shape=jax.ShapeDtypeStruct((BS, 80), jnp.float32),
    )(flat, wt, rb)


def _sc_combine(flat, eb, meta):
    BS = flat.shape[0]
    tok_per_w = BS // _TSH
    n_chunks = tok_per_w // _TCHUNK
    mesh = plsc.VectorSubcoreMesh(core_axis_name="c", subcore_axis_name="s")

    @functools.partial(
        pl.kernel, mesh=mesh,
        out_type=jax.ShapeDtypeStruct((BS, _H), jnp.float32),
        scratch_types=[
            pltpu.VMEM((_E * _HL,), jnp.float32),
            pltpu.VMEM((_TCHUNK, _HL), jnp.float32),
            pltpu.VMEM((_TCHUNK, _HL), jnp.float32),
            pltpu.VMEM((_TCHUNK, 80), jnp.float32),
        ],
        compiler_params=pltpu.CompilerParams(needs_layout_passes=False),
    )
    def k(x_hbm, eb_hbm, meta_hbm, out_hbm, ebt, xb, ob, mr):
        wid = lax.axis_index("s") * 2 + lax.axis_index("c")
        tsh = wid // _HSH
        hsh = wid % _HSH
        t0 = tsh * tok_per_w
        c0 = hsh * _HL

        for e in range(_E):
            pltpu.sync_copy(eb_hbm.at[e, pl.ds(c0, _HL)],
                            ebt.at[pl.ds(e * _HL, _HL)])

        iota = lax.iota(jnp.int32, 16)

        def chunk_body(ci, carry):
            tb = t0 + ci * _TCHUNK
            pltpu.sync_copy(x_hbm.at[pl.ds(tb, _TCHUNK), pl.ds(c0, _HL)],
                            xb)
            pltpu.sync_copy(meta_hbm.at[pl.ds(tb, _TCHUNK), :], mr)

            def tok_body(t, carry1):
                ws_s = mr[t, pl.ds(0, 16)]
                w0_s = mr[t, pl.ds(16, 16)]
                w1_s = mr[t, pl.ds(32, 16)]
                b0 = mr[t, pl.ds(48, 16)].astype(jnp.int32) + iota
                b1 = mr[t, pl.ds(64, 16)].astype(jnp.int32) + iota

                def h_body(hb, carry2):
                    h = hb * 16
                    x_v = xb[t, pl.ds(h, 16)]
                    b0_v = plsc.load_gather(ebt, [b0 + h])
                    b1_v = plsc.load_gather(ebt, [b1 + h])
                    ob[t, pl.ds(h, 16)] = (
                        ws_s * x_v + w0_s * b0_v + w1_s * b1_v)
                    return carry2

                lax.fori_loop(0, _HL // 16, h_body, 0, unroll=4)
                return carry1

            lax.fori_loop(0, _TCHUNK, tok_body, 0)

            pltpu.sync_copy(
                ob, out_hbm.at[pl.ds(tb, _TCHUNK), pl.ds(c0, _HL)])
            return carry

        lax.fori_loop(0, n_chunks, chunk_body, 0)

    return k(flat, eb, meta)


def kernel(hidden_states, router_weight, router_bias, expert_bias):
    B, S, H = hidden_states.shape
    BS = B * S
    flat = hidden_states.reshape(BS, H)
    meta = _router(flat, router_weight, router_bias)
    out = _sc_combine(flat, expert_bias, meta)
    return out.reshape(B, S, H)


# speedup vs baseline: 7.8476x; 7.8476x over previous
import jax
import jax.numpy as jnp
from jax.experimental import pallas as pl
from jax.experimental.pallas import tpu as pltpu

_E = 64
_ROWS = 1024


def _fused_body(x_ref, w_ref, rb_ref, eb_ref, out_ref):
    x = x_ref[...]
    logits = jax.lax.dot_general(
        x, w_ref[...], (((1,), (1,)), ((), ())),
        preferred_element_type=jnp.float32)
    logits = logits + rb_ref[...]

    ml1 = jnp.max(logits, axis=-1, keepdims=True)
    lm = jnp.where(logits == ml1, -jnp.inf, logits)
    ml2 = jnp.max(lm, axis=-1, keepdims=True)

    ex = jnp.exp(logits - ml1)
    r = 1.0 / jnp.sum(ex, axis=-1, keepdims=True)

    s_masked = jnp.where(logits >= ml2, ex, 0.0) * r
    wsum = (1.0 + jnp.exp(ml2 - ml1)) * r

    comb = jnp.dot(s_masked.astype(jnp.bfloat16), eb_ref[...],
                   preferred_element_type=jnp.float32)
    out_ref[...] = wsum * x + comb


def kernel(hidden_states, router_weight, router_bias, expert_bias):
    B, S, H = hidden_states.shape
    BS = B * S
    flat = hidden_states.reshape(BS, H)
    rb = router_bias.reshape(1, _E)
    eb16 = expert_bias.astype(jnp.bfloat16)

    out = pl.pallas_call(
        _fused_body,
        grid=(BS // _ROWS,),
        in_specs=[
            pl.BlockSpec((_ROWS, H), lambda i: (i, 0)),
            pl.BlockSpec((_E, H), lambda i: (0, 0)),
            pl.BlockSpec((1, _E), lambda i: (0, 0)),
            pl.BlockSpec((_E, H), lambda i: (0, 0)),
        ],
        out_specs=pl.BlockSpec((_ROWS, H), lambda i: (i, 0)),
        out_shape=jax.ShapeDtypeStruct((BS, H), jnp.float32),
        compiler_params=pltpu.CompilerParams(
            dimension_semantics=("parallel",)),
    )(flat, router_weight, rb, eb16)
    return out.reshape(B, S, H)
